# baseline (speedup 1.0000x reference)
import functools

import jax
import jax.numpy as jnp
from jax import lax
from jax.experimental import pallas as pl
from jax.experimental.pallas import tpu as pltpu
from jax.experimental.pallas import tpu_sc as plsc

_H = 64
_W = 64
_HW = _H * _W
_K = 16
_BB = 16
_CH = 16


def _tc_body(y_ref, r1_ref, r2_ref, fmb_ref, row_ref, gf_ref):
    yb = y_ref[...]
    m = jnp.max(yb, axis=-1, keepdims=True)
    lane = lax.broadcasted_iota(jnp.int32, yb.shape, 2)
    cand = jnp.where(yb == m, lane, _HW)
    idx = jnp.min(cand, axis=-1, keepdims=True)
    idx = jnp.where(m > 0.0, idx, 0)
    idx2 = jnp.reshape(idx, (_BB * _K, 1))
    px = idx2 & (_W - 1)
    py = idx2 >> 6
    row_ref[...] = jnp.reshape(px * _W + py, (_BB, _K))
    c64 = lax.broadcasted_iota(jnp.int32, (_BB * _K, _W), 1)
    oh_y = (py == c64).astype(jnp.float32)
    oh_x = (px == c64).astype(jnp.float32)
    y1 = jnp.dot(oh_y, r1_ref[...], preferred_element_type=jnp.float32)
    x1 = jnp.dot(oh_x, r2_ref[...], preferred_element_type=jnp.float32)
    gt = y1 * x1
    gf = jnp.dot(fmb_ref[...], gt, preferred_element_type=jnp.float32)
    gf = jnp.clip(gf, 0.0, 1.0)
    for b in range(_BB):
        gf_ref[b, :, :] = gf[b * _K:(b + 1) * _K, :]


def _tc_stage(yf, r1, r2, fmb, B):
    grid = (B // _BB,)
    return pl.pallas_call(
        _tc_body,
        grid=grid,
        in_specs=[
            pl.BlockSpec((_BB, _K, _HW), lambda i: (i, 0, 0)),
            pl.BlockSpec((_W, _HW), lambda i: (0, 0)),
            pl.BlockSpec((_W, _HW), lambda i: (0, 0)),
            pl.BlockSpec((_BB * _K, _BB * _K), lambda i: (0, 0)),
        ],
        out_specs=[
            pl.BlockSpec((_BB, _K), lambda i: (i, 0)),
            pl.BlockSpec((_BB, _K, _HW), lambda i: (i, 0, 0)),
        ],
        out_shape=[
            jax.ShapeDtypeStruct((B, _K), jnp.int32),
            jax.ShapeDtypeStruct((B, _K, _HW), jnp.float32),
        ],
    )(yf, r1, r2, fmb)


def _make_sc_gather(n_rows):
    info = plsc.get_sparse_core_info()
    nw = info.num_cores * info.num_subcores
    per_w = n_rows // nw
    n_ch = per_w // _CH
    mesh = plsc.VectorSubcoreMesh(core_axis_name="c", subcore_axis_name="s")

    @functools.partial(
        pl.kernel,
        mesh=mesh,
        out_type=jax.ShapeDtypeStruct((n_rows, _HW), jnp.float32),
        scratch_types=[
            pltpu.VMEM((_CH,), jnp.int32),
            pltpu.VMEM((_CH, _HW), jnp.float32),
            pltpu.SemaphoreType.DMA,
        ],
    )
    def sc_gather(table_hbm, idx_hbm, out_hbm, idx_v, rows_v, sem):
        wid = lax.axis_index("s") * info.num_cores + lax.axis_index("c")
        base = wid * per_w
        for c in range(n_ch):
            off = base + c * _CH
            pltpu.sync_copy(idx_hbm.at[pl.ds(off, _CH)], idx_v)
            pltpu.async_copy(table_hbm.at[idx_v], rows_v, sem).wait()
            pltpu.sync_copy(rows_v, out_hbm.at[pl.ds(off, _CH)])

    return sc_gather


def kernel(y, heatmaps, false_matrix):
    B, K, H, W = y.shape
    yf = y.reshape(B, K, H * W)
    g = heatmaps[:, 0, 0, :]
    r1 = jnp.repeat(g, W, axis=1)
    r2 = jnp.tile(g, (1, H))
    fmb = jnp.kron(jnp.eye(_BB, dtype=jnp.float32), false_matrix.T)
    rows, gf = _tc_stage(yf, r1, r2, fmb, B)
    table = heatmaps.reshape(64 * 64, H * W)
    idx_flat = rows.reshape(B * K)
    gt = _make_sc_gather(B * K)(table, idx_flat)
    return gt.reshape(B, K, H, W), gf.reshape(B, K, H, W)
